# baseline (speedup 1.0000x reference)
import functools

import jax
import jax.numpy as jnp
from jax import lax
from jax.experimental import pallas as pl
from jax.experimental.pallas import tpu as pltpu
from jax.experimental.pallas import tpu_sc as plsc

NUM_CORES = 2
NUM_SUBCORES = 16
LANES = 16
NW = NUM_CORES * NUM_SUBCORES

_B = 16384
_H = 8
_D = 32
_JB = _B // 128
_JW = _JB // NW
_CHUNK = _H * 128


@functools.lru_cache(maxsize=None)
def _build_lookup():
    mesh = plsc.VectorSubcoreMesh(core_axis_name="c", subcore_axis_name="s")
    n_oct = _D // 8

    @functools.partial(
        pl.kernel,
        out_type=jax.ShapeDtypeStruct((_H, n_oct, _JB, 8, 128), jnp.float32),
        mesh=mesh,
        compiler_params=pltpu.CompilerParams(
            use_tc_tiling_on_sc=False, needs_layout_passes=False),
        scratch_types=[
            pltpu.VMEM((_JW, _H, 128), jnp.int32),
            pltpu.VMEM((_JW * _CHUNK,), jnp.int32),
            pltpu.VMEM((_CHUNK, _D), jnp.float32),
            pltpu.VMEM((_CHUNK, _D), jnp.float32),
            pltpu.VMEM((_H, n_oct, 1, 8, 128), jnp.float32),
            pltpu.VMEM((LANES,), jnp.int32),
            pltpu.SemaphoreType.DMA,
            pltpu.SemaphoreType.DMA,
        ],
    )
    def lookup(ids_hbm, offs_hbm, table_hbm, out_hbm,
               idx_v, shift_v, rows_a, rows_b, stg_v, offs_v, ga, gb):
        wid = lax.axis_index("s") * NUM_CORES + lax.axis_index("c")
        jbase = wid * _JW

        pltpu.sync_copy(ids_hbm.at[pl.ds(jbase, _JW)], idx_v)
        for rep in range(LANES // _H):
            pltpu.sync_copy(offs_hbm, offs_v.at[pl.ds(rep * _H, _H)])

        lane = lax.iota(jnp.int32, LANES)

        def add_body(i, carry):
            j = i >> 6
            h = (i >> 3) & 7
            lb = i & 7
            off = plsc.load_gather(offs_v, [jnp.broadcast_to(h, (LANES,))])
            shift_v[pl.ds(i * LANES, LANES)] = idx_v[j, h, pl.ds(lb * LANES, LANES)] + off
            return carry

        lax.fori_loop(0, _JW * _CHUNK // LANES, add_body, 0)

        rows = (rows_a, rows_b)
        sems = (ga, gb)

        def gather(j):
            return pltpu.make_async_copy(
                table_hbm.at[shift_v.at[pl.ds(j * _CHUNK, _CHUNK)]],
                rows[j % 2], sems[j % 2])

        def extract(j):
            rows_j = rows[j % 2]

            def body(i, carry):
                h = i >> 5
                o = (i >> 3) & 3
                r = i & 7
                d = i & 31
                dvec = jnp.broadcast_to(jnp.int32(d), (LANES,))
                for lb in range(8):
                    p0 = h * 128 + lb * LANES
                    vals = plsc.load_gather(rows_j, [p0 + lane, dvec])
                    stg_v[h, o, 0, r, pl.ds(lb * LANES, LANES)] = vals
                return carry

            lax.fori_loop(0, _H * _D, body, 0)

        gather(0).start()
        for j in range(_JW):
            if j + 1 < _JW:
                gather(j + 1).start()
            gather(j).wait()
            extract(j)
            for h in range(_H):
                for o in range(n_oct):
                    pltpu.sync_copy(
                        stg_v.at[h, o],
                        out_hbm.at[h, o, pl.ds(jbase + j, 1)])

    return lookup


def kernel(input_ids, offsets, table):
    b, h = input_ids.shape
    _, d = table.shape
    assert (b, h, d) == (_B, _H, _D)
    ids3 = input_ids.reshape(_JB, 128, _H).transpose(0, 2, 1)
    out5 = _build_lookup()(ids3, offsets, table)
    return out5.transpose(2, 4, 0, 1, 3).reshape(_B, _H, _D)


# speedup vs baseline: 1.0206x; 1.0206x over previous
import functools

import jax
import jax.numpy as jnp
from jax import lax
from jax.experimental import pallas as pl
from jax.experimental.pallas import tpu as pltpu
from jax.experimental.pallas import tpu_sc as plsc

NUM_CORES = 2
NUM_SUBCORES = 16
LANES = 16
NW = NUM_CORES * NUM_SUBCORES

_B = 16384
_H = 8
_D = 32
_JB = _B // 128
_JW = _JB // NW
_HH = _H // 2
_CHUNK = _HH * 128
_NC = _JW * 2


@functools.lru_cache(maxsize=None)
def _build_lookup():
    mesh = plsc.VectorSubcoreMesh(core_axis_name="c", subcore_axis_name="s")
    n_oct = _D // 8

    @functools.partial(
        pl.kernel,
        out_type=jax.ShapeDtypeStruct((_H, n_oct, _JB, 8, 128), jnp.float32),
        mesh=mesh,
        compiler_params=pltpu.CompilerParams(
            use_tc_tiling_on_sc=False, needs_layout_passes=False),
        scratch_types=[
            pltpu.VMEM((_JW, _H, 128), jnp.int32),
            pltpu.VMEM((_NC, _CHUNK), jnp.int32),
            pltpu.VMEM((_CHUNK, _D), jnp.float32),
            pltpu.VMEM((_CHUNK, _D), jnp.float32),
            pltpu.VMEM((_H, n_oct, 1, 8, 128), jnp.float32),
            pltpu.VMEM((_H, n_oct, 1, 8, 128), jnp.float32),
            pltpu.VMEM((LANES,), jnp.int32),
            pltpu.SemaphoreType.DMA,
            pltpu.SemaphoreType.DMA,
            pltpu.SemaphoreType.DMA,
            pltpu.SemaphoreType.DMA,
        ],
    )
    def lookup(ids_hbm, offs_hbm, table_hbm, out_hbm,
               idx_v, shift_v, rows_a, rows_b, stg_a, stg_b, offs_v,
               ga, gb, sa, sb):
        rows_bufs = (rows_a, rows_b)
        stg_bufs = (stg_a, stg_b)
        wid = lax.axis_index("s") * NUM_CORES + lax.axis_index("c")
        jbase = wid * _JW

        pltpu.sync_copy(ids_hbm.at[pl.ds(jbase, _JW)], idx_v)
        for rep in range(LANES // _H):
            pltpu.sync_copy(offs_hbm, offs_v.at[pl.ds(rep * _H, _H)])

        lane = lax.iota(jnp.int32, LANES)

        offs_full = offs_v[...]
        for h in range(_H):
            off = jnp.sum(jnp.where(lane == h, offs_full, 0))

            def add_body(i, carry, h=h, off=off):
                j = i >> 3
                lb = i & 7
                shift_v[j * 2 + (h // _HH), pl.ds((h % _HH) * 128 + lb * LANES, LANES)] = (
                    idx_v[j, h, pl.ds(lb * LANES, LANES)] + off)
                return carry

            lax.fori_loop(0, _JW * 8, add_body, 0)

        gsems = (ga, gb)
        ssems = (sa, sb)
        dconst = [jnp.broadcast_to(jnp.int32(d), (LANES,)) for d in range(_D)]

        def gather(c):
            return pltpu.make_async_copy(
                table_hbm.at[shift_v.at[c]],
                rows_bufs[c % 2], gsems[c % 2])

        def extract(c):
            half = c % 2
            j = c // 2
            rows_c = rows_bufs[half]
            stg_c = stg_bufs[j % 2]

            def body(i, carry):
                hl = i >> 3
                lb = i & 7
                h = half * _HH + hl
                idx0 = hl * 128 + lb * LANES + lane
                for o in range(n_oct):
                    for r in range(8):
                        vals = plsc.load_gather(rows_c, [idx0, dconst[o * 8 + r]])
                        stg_c[h, o, 0, r, pl.ds(lb * LANES, LANES)] = vals
                return carry

            lax.fori_loop(0, _HH * 8, body, 0)

        def stores(j):
            return [
                pltpu.make_async_copy(
                    stg_bufs[j % 2].at[h, o],
                    out_hbm.at[h, o, pl.ds(jbase + j, 1)], ssems[j % 2])
                for h in range(_H) for o in range(n_oct)
            ]

        gather(0).start()
        for c in range(_NC):
            if c + 1 < _NC:
                gather(c + 1).start()
            gather(c).wait()
            if c % 2 == 0 and c >= 4:
                for s in stores(c // 2 - 2):
                    s.wait()
            extract(c)
            if c % 2 == 1:
                for s in stores(c // 2):
                    s.start()
        for j in (_NC // 2 - 2, _NC // 2 - 1):
            for s in stores(j):
                s.wait()

    return lookup


def kernel(input_ids, offsets, table):
    b, h = input_ids.shape
    _, d = table.shape
    assert (b, h, d) == (_B, _H, _D)
    ids3 = input_ids.reshape(_JB, 128, _H).transpose(0, 2, 1)
    out5 = _build_lookup()(ids3, offsets, table)
    return out5.transpose(2, 4, 0, 1, 3).reshape(_B, _H, _D)


# speedup vs baseline: 1.0679x; 1.0463x over previous
import functools

import jax
import jax.numpy as jnp
from jax import lax
from jax.experimental import pallas as pl
from jax.experimental.pallas import tpu as pltpu
from jax.experimental.pallas import tpu_sc as plsc

NUM_CORES = 2
NUM_SUBCORES = 16
LANES = 16
NW = NUM_CORES * NUM_SUBCORES

_B = 16384
_H = 8
_D = 32
_JB = _B // 128
_JW = _JB // NW
_HH = _H // 2
_CHUNK = _HH * 128
_NC = _JW * 2


@functools.lru_cache(maxsize=None)
def _build_lookup():
    mesh = plsc.VectorSubcoreMesh(core_axis_name="c", subcore_axis_name="s")
    n_oct = _D // 8

    @functools.partial(
        pl.kernel,
        out_type=jax.ShapeDtypeStruct((_H * n_oct * _JB * 8 * 128,), jnp.float32),
        mesh=mesh,
        compiler_params=pltpu.CompilerParams(
            use_tc_tiling_on_sc=False, needs_layout_passes=False),
        scratch_types=[
            pltpu.VMEM((_JW, _H, 128), jnp.int32),
            pltpu.VMEM((_NC, _CHUNK), jnp.int32),
            pltpu.VMEM((_CHUNK, _D), jnp.float32),
            pltpu.VMEM((_CHUNK, _D), jnp.float32),
            pltpu.VMEM((_H * n_oct * 8 * 128,), jnp.float32),
            pltpu.VMEM((_H * n_oct * 8 * 128,), jnp.float32),
            pltpu.VMEM((LANES,), jnp.int32),
            pltpu.SemaphoreType.DMA,
            pltpu.SemaphoreType.DMA,
            pltpu.SemaphoreType.DMA,
            pltpu.SemaphoreType.DMA,
        ],
    )
    def lookup(ids_hbm, offs_hbm, table_hbm, out_hbm,
               idx_v, shift_v, rows_a, rows_b, stg_a, stg_b, offs_v,
               ga, gb, sa, sb):
        rows_bufs = (rows_a, rows_b)
        stg_bufs = (stg_a, stg_b)
        wid = lax.axis_index("s") * NUM_CORES + lax.axis_index("c")
        jbase = wid * _JW

        pltpu.sync_copy(ids_hbm.at[pl.ds(jbase, _JW)], idx_v)
        for rep in range(LANES // _H):
            pltpu.sync_copy(offs_hbm, offs_v.at[pl.ds(rep * _H, _H)])

        lane = lax.iota(jnp.int32, LANES)

        offs_full = offs_v[...]
        for h in range(_H):
            off = jnp.sum(jnp.where(lane == h, offs_full, 0))

            def add_body(i, carry, h=h, off=off):
                j = i >> 3
                lb = i & 7
                shift_v[j * 2 + (h // _HH), pl.ds((h % _HH) * 128 + lb * LANES, LANES)] = (
                    idx_v[j, h, pl.ds(lb * LANES, LANES)] + off)
                return carry

            lax.fori_loop(0, _JW * 8, add_body, 0)

        gsems = (ga, gb)
        ssems = (sa, sb)
        p1 = (lane >> 3) * 1024 + (lane & 7) * 128
        p2 = p1 + 2048

        def gather(c):
            return pltpu.make_async_copy(
                table_hbm.at[shift_v.at[c]],
                rows_bufs[c % 2], gsems[c % 2])

        def extract(c):
            half = c % 2
            j = c // 2
            rows_c = rows_bufs[half]
            stg_c = stg_bufs[j % 2]

            def body(i, carry):
                for u in range(2):
                    p = i * 2 + u
                    base = (half * _HH + (p >> 7)) * 4096 + (p & 127)
                    plsc.store_scatter(stg_c, [p1 + base], rows_c[p, pl.ds(0, LANES)])
                    plsc.store_scatter(stg_c, [p2 + base], rows_c[p, pl.ds(LANES, LANES)])
                return carry

            lax.fori_loop(0, _CHUNK // 2, body, 0)

        def stores(j):
            return [
                pltpu.make_async_copy(
                    stg_bufs[j % 2].at[pl.ds((h * n_oct + o) * 1024, 1024)],
                    out_hbm.at[pl.ds(((h * n_oct + o) * _JB + jbase + j) * 1024, 1024)],
                    ssems[j % 2])
                for h in range(_H) for o in range(n_oct)
            ]

        gather(0).start()
        for c in range(_NC):
            if c + 1 < _NC:
                gather(c + 1).start()
            gather(c).wait()
            if c % 2 == 0 and c >= 4:
                for s in stores(c // 2 - 2):
                    s.wait()
            extract(c)
            if c % 2 == 1:
                for s in stores(c // 2):
                    s.start()
        for j in (_NC // 2 - 2, _NC // 2 - 1):
            for s in stores(j):
                s.wait()

    return lookup


def kernel(input_ids, offsets, table):
    b, h = input_ids.shape
    _, d = table.shape
    assert (b, h, d) == (_B, _H, _D)
    ids3 = input_ids.reshape(_JB, 128, _H).transpose(0, 2, 1)
    out_flat = _build_lookup()(ids3, offsets, table)
    out5 = out_flat.reshape(_H, _D // 8, _JB, 8, 128)
    return out5.transpose(2, 4, 0, 1, 3).reshape(_B, _H, _D)
